# baseline (speedup 1.0000x reference)
import functools

import jax
import jax.numpy as jnp
from jax import lax
from jax.experimental import pallas as pl
from jax.experimental.pallas import tpu as pltpu
from jax.experimental.pallas import tpu_sc as plsc

N = 10000
E = 320000
D = 128

NC = 2
NS = 16
NW = NC * NS

EPT = E // NW
CH = 80
NCH = EPT // CH
NPAD = 10240
RPT = NPAD // NS
OC = RPT // CH

_BN = 1000


def _agg_kernel_entry(table, srcs, dsts, zeros, out,
                      sb0, sb1, db0, db1, r0, r1, acc,
                      sg0, sg1, sw0, sw1, ss0, ss1, sd0, sd1):
    sbuf = [sb0, sb1]
    dbuf = [db0, db1]
    rows = [r0, r1]
    semg = [sg0, sg1]
    semw = [sw0, sw1]
    sems = [ss0, ss1]
    semd = [sd0, sd1]

    cid = lax.axis_index("c")
    sid = lax.axis_index("s")
    tid = cid * NS + sid
    ebase = tid * EPT

    def f_src(j, b):
        off = pl.multiple_of(ebase + j * CH, 8)
        pltpu.async_copy(srcs.at[pl.ds(off, CH)], sbuf[b], sems[b])

    def f_dst(j, b):
        off = pl.multiple_of(ebase + j * CH, 8)
        pltpu.async_copy(dsts.at[pl.ds(off, CH)], dbuf[b], semd[b])

    def f_gather(b):
        pltpu.async_copy(table.at[sbuf[b]], rows[b], semg[b])

    def f_scatter(b):
        pltpu.async_copy(rows[b], acc.at[dbuf[b]], semw[b], add=True)

    def wait_g(b):
        pltpu.make_async_copy(table.at[pl.ds(0, CH)], rows[b], semg[b]).wait()

    def wait_w(b):
        pltpu.make_async_copy(table.at[pl.ds(0, CH)], rows[b], semw[b]).wait()

    def wait_s(b):
        pltpu.make_async_copy(srcs.at[pl.ds(0, CH)], sbuf[b], sems[b]).wait()

    def wait_d(b):
        pltpu.make_async_copy(srcs.at[pl.ds(0, CH)], sbuf[b], semd[b]).wait()

    def slab(k):
        return pl.multiple_of(sid * RPT + k * CH, 8)

    with jax.named_scope("agg_zero"):
        pltpu.sync_copy(zeros, r0)
        for k in range(OC):
            pltpu.async_copy(r0, acc.at[pl.ds(slab(k), CH)], ss0)
        for k in range(OC):
            pltpu.make_async_copy(zeros, r1, ss0).wait()

    plsc.subcore_barrier()

    def slot(j, b, waitw=True, dosrc=True, dog=True):
        b1 = (b + 1) % 2
        wait_g(b)
        wait_d(b)
        f_scatter(b)
        if dosrc:
            f_src(j + 2, b)
        if dog:
            wait_s(b1)
            if waitw:
                wait_w(b1)
            f_gather(b1)
            f_dst(j + 1, b1)

    with jax.named_scope("agg_edges"):
        f_src(0, 0)
        f_src(1, 1)
        f_dst(0, 0)
        wait_s(0)
        f_gather(0)
        slot(0, 0, waitw=False)

        def body(t, carry):
            j = 2 * t + 1
            slot(j, 1)
            slot(j + 1, 0)
            return carry

        lax.fori_loop(0, (NCH - 3) // 2, body, 0)
        slot(NCH - 2, 1, dosrc=False)
        slot(NCH - 1, 0, dosrc=False, dog=False)
        wait_w(0)
        wait_w(1)
        plsc.subcore_barrier()

    with jax.named_scope("agg_out"):
        def f_rd(k, b):
            pltpu.async_copy(acc.at[pl.ds(slab(k), CH)], rows[b], semg[b])

        def f_wr(k, b):
            pltpu.async_copy(rows[b], out.at[cid, pl.ds(slab(k), CH)],
                             semw[b])

        f_rd(0, 0)
        f_rd(1, 1)
        for k in range(OC):
            b = k % 2
            wait_g(b)
            f_wr(k, b)
            if k + 2 < OC:
                wait_w(b)
                f_rd(k + 2, b)
        wait_w(0)
        wait_w(1)


@jax.jit
def _aggregate(table, srcs, dsts, zeros):
    mesh = plsc.VectorSubcoreMesh(core_axis_name="c", subcore_axis_name="s")
    k = functools.partial(
        pl.kernel,
        mesh=mesh,
        out_type=jax.ShapeDtypeStruct((NC, NPAD, D), jnp.float32),
        scratch_types=[
            pltpu.VMEM((CH,), jnp.int32),
            pltpu.VMEM((CH,), jnp.int32),
            pltpu.VMEM((CH,), jnp.int32),
            pltpu.VMEM((CH,), jnp.int32),
            pltpu.VMEM((CH, D), jnp.float32),
            pltpu.VMEM((CH, D), jnp.float32),
            pltpu.VMEM_SHARED((NPAD, D), jnp.float32),
        ] + [pltpu.SemaphoreType.DMA] * 8,
    )(_agg_kernel_entry)
    return k(table, srcs, dsts, zeros)


def _pre_body(x_ref, wint_ref, bin_ref, w0_ref, h_ref, hw_ref):
    h = jnp.dot(x_ref[...], wint_ref[...],
                preferred_element_type=jnp.float32) + bin_ref[...]
    h_ref[...] = h
    hw_ref[...] = jnp.dot(h, w0_ref[...], preferred_element_type=jnp.float32)


def _ln(h, g, be):
    mu = jnp.mean(h, axis=-1, keepdims=True)
    var = jnp.mean((h - mu) ** 2, axis=-1, keepdims=True)
    return (h - mu) * lax.rsqrt(var + 1e-5) * g + be


def _mid_body(p_ref, b_ref, res_ref, g_ref, be_ref, wn_ref, h_ref, hw_ref):
    s = p_ref[0] + p_ref[1] + b_ref[...]
    h = jnp.maximum(s, 0.0) + res_ref[...]
    hn = _ln(h, g_ref[...], be_ref[...])
    h_ref[...] = hn
    hw_ref[...] = jnp.dot(hn, wn_ref[...], preferred_element_type=jnp.float32)


def _fin_body(p_ref, b_ref, res_ref, g_ref, be_ref, wot_ref, bo_ref, o_ref):
    s = p_ref[0] + p_ref[1] + b_ref[...]
    h = jnp.maximum(s, 0.0) + res_ref[...]
    hn = _ln(h, g_ref[...], be_ref[...])
    o_ref[...] = jnp.dot(hn, wot_ref[...],
                         preferred_element_type=jnp.float32) + bo_ref[...]


_row_spec = pl.BlockSpec((_BN, D), lambda i: (i, 0))
_mat_spec = pl.BlockSpec((D, D), lambda i: (0, 0))
_vec_spec = pl.BlockSpec((1, D), lambda i: (0, 0))
_par_spec = pl.BlockSpec((NC, _BN, D), lambda i: (0, i, 0))
_out2 = [jax.ShapeDtypeStruct((N, D), jnp.float32)] * 2
_out1 = jax.ShapeDtypeStruct((N, D), jnp.float32)


@jax.jit
def _pre(x, wint, bin_, w0):
    return pl.pallas_call(
        _pre_body,
        grid=(N // _BN,),
        in_specs=[_row_spec, _mat_spec, _vec_spec, _mat_spec],
        out_specs=[_row_spec, _row_spec],
        out_shape=_out2,
    )(x, wint, bin_, w0)


@jax.jit
def _mid(p, b, res, g, be, wn):
    return pl.pallas_call(
        _mid_body,
        grid=(N // _BN,),
        in_specs=[_par_spec, _vec_spec, _row_spec, _vec_spec, _vec_spec,
                  _mat_spec],
        out_specs=[_row_spec, _row_spec],
        out_shape=_out2,
    )(p, b, res, g, be, wn)


@jax.jit
def _fin(p, b, res, g, be, wot, bo):
    return pl.pallas_call(
        _fin_body,
        grid=(N // _BN,),
        in_specs=[_par_spec, _vec_spec, _row_spec, _vec_spec, _vec_spec,
                  _mat_spec, _vec_spec],
        out_specs=_row_spec,
        out_shape=_out1,
    )(p, b, res, g, be, wot, bo)


def kernel(node_features, W_in, b_in, W0, b0, g0, be0, W1, b1, g1, be1,
           W2, b2, g2, be2, W_out, b_out, edge_index):
    srcs = edge_index[0]
    dsts = edge_index[1]
    zeros = jnp.zeros((CH, D), jnp.float32)

    r2 = lambda v: v.reshape(1, D)

    h, hw = _pre(node_features, W_in.T, r2(b_in), W0)

    p = _aggregate(hw, srcs, dsts, zeros)
    h, hw = _mid(p, r2(b0), h, r2(g0), r2(be0), W1)

    p = _aggregate(hw, srcs, dsts, zeros)
    h, hw = _mid(p, r2(b1), h, r2(g1), r2(be1), W2)

    p = _aggregate(hw, srcs, dsts, zeros)
    return _fin(p, r2(b2), h, r2(g2), r2(be2), W_out.T, r2(b_out))


# speedup vs baseline: 1.2195x; 1.2195x over previous
import functools

import jax
import jax.numpy as jnp
from jax import lax
from jax.experimental import pallas as pl
from jax.experimental.pallas import tpu as pltpu
from jax.experimental.pallas import tpu_sc as plsc

N = 10000
E = 320000
D = 128

NC = 2
NS = 16
NW = NC * NS

EPT = E // NW
CH = 80
NCH = EPT // CH
NPAD = 10240
RPT = NPAD // NS
OC = RPT // CH

_BN = 1000


def _agg_kernel_entry(table, srcs, dsts, zeros, out,
                      sb0, sb1, dst_v, r0, r1, acc,
                      sg0, sg1, ss0, ss1):
    sbuf = [sb0, sb1]
    rows = [r0, r1]
    semg = [sg0, sg1]
    sems = [ss0, ss1]

    cid = lax.axis_index("c")
    sid = lax.axis_index("s")
    tid = cid * NS + sid
    ebase = tid * EPT

    def f_src(j, b):
        off = pl.multiple_of(ebase + j * CH, 8)
        pltpu.async_copy(srcs.at[pl.ds(off, CH)], sbuf[b], sems[b])

    def f_gather(b):
        pltpu.async_copy(table.at[sbuf[b]], rows[b], semg[b])

    def wait_g(b):
        pltpu.make_async_copy(table.at[pl.ds(0, CH)], rows[b], semg[b]).wait()

    def wait_s(b):
        pltpu.make_async_copy(srcs.at[pl.ds(0, CH)], sbuf[b], sems[b]).wait()

    def slab(k):
        return pl.multiple_of(sid * RPT + k * CH, 8)

    with jax.named_scope("agg_zero"):
        pltpu.sync_copy(zeros, r0)
        for k in range(OC):
            pltpu.async_copy(r0, acc.at[pl.ds(slab(k), CH)], ss0)
        for k in range(OC):
            pltpu.make_async_copy(zeros, r1, ss0).wait()

    pltpu.sync_copy(dsts.at[tid], dst_v)
    plsc.subcore_barrier()

    with jax.named_scope("agg_edges"):
        f_src(0, 0)
        f_src(1, 1)
        wait_s(0)
        f_gather(0)
        wait_s(1)
        f_gather(1)

        def body(t, carry):
            j0 = 2 * t
            wait_g(0)
            f_src(j0 + 2, 0)
            pltpu.sync_copy(r0, acc.at[dst_v.at[j0]], add=True)
            wait_s(0)
            f_gather(0)

            wait_g(1)

            @pl.when(t < NCH // 2 - 1)
            def _():
                f_src(j0 + 3, 1)
                pltpu.sync_copy(r1, acc.at[dst_v.at[j0 + 1]], add=True)
                wait_s(1)
                f_gather(1)

            @pl.when(t >= NCH // 2 - 1)
            def _():
                pltpu.sync_copy(r1, acc.at[dst_v.at[j0 + 1]], add=True)

            return carry

        lax.fori_loop(0, NCH // 2, body, 0)
        wait_g(0)
        pltpu.sync_copy(r0, acc.at[dst_v.at[NCH - 1]], add=True)
        plsc.subcore_barrier()

    with jax.named_scope("agg_out"):
        def f_rd(k, b):
            pltpu.async_copy(acc.at[pl.ds(slab(k), CH)], rows[b], semg[b])

        def f_wr(k, b):
            pltpu.async_copy(rows[b], out.at[cid, pl.ds(slab(k), CH)],
                             sems[b])

        def wait_wr(b):
            pltpu.make_async_copy(table.at[pl.ds(0, CH)], rows[b],
                                  sems[b]).wait()

        f_rd(0, 0)
        f_rd(1, 1)
        for k in range(OC):
            b = k % 2
            wait_g(b)
            f_wr(k, b)
            if k + 2 < OC:
                wait_wr(b)
                f_rd(k + 2, b)
        wait_wr(0)
        wait_wr(1)


@jax.jit
def _aggregate(table, srcs, dsts, zeros):
    mesh = plsc.VectorSubcoreMesh(core_axis_name="c", subcore_axis_name="s")
    k = functools.partial(
        pl.kernel,
        mesh=mesh,
        out_type=jax.ShapeDtypeStruct((NC, NPAD, D), jnp.float32),
        scratch_types=[
            pltpu.VMEM((CH,), jnp.int32),
            pltpu.VMEM((CH,), jnp.int32),
            pltpu.VMEM((NCH, CH), jnp.int32),
            pltpu.VMEM((CH, D), jnp.float32),
            pltpu.VMEM((CH, D), jnp.float32),
            pltpu.VMEM_SHARED((NPAD, D), jnp.float32),
        ] + [pltpu.SemaphoreType.DMA] * 4,
    )(_agg_kernel_entry)
    return k(table, srcs, dsts, zeros)


def _pre_body(x_ref, wint_ref, bin_ref, w0_ref, h_ref, hw_ref):
    h = jnp.dot(x_ref[...], wint_ref[...],
                preferred_element_type=jnp.float32) + bin_ref[...]
    h_ref[...] = h
    hw_ref[...] = jnp.dot(h, w0_ref[...], preferred_element_type=jnp.float32)


def _ln(h, g, be):
    mu = jnp.mean(h, axis=-1, keepdims=True)
    var = jnp.mean((h - mu) ** 2, axis=-1, keepdims=True)
    return (h - mu) * lax.rsqrt(var + 1e-5) * g + be


def _mid_body(p_ref, b_ref, res_ref, g_ref, be_ref, wn_ref, h_ref, hw_ref):
    s = p_ref[0] + p_ref[1] + b_ref[...]
    h = jnp.maximum(s, 0.0) + res_ref[...]
    hn = _ln(h, g_ref[...], be_ref[...])
    h_ref[...] = hn
    hw_ref[...] = jnp.dot(hn, wn_ref[...], preferred_element_type=jnp.float32)


def _fin_body(p_ref, b_ref, res_ref, g_ref, be_ref, wot_ref, bo_ref, o_ref):
    s = p_ref[0] + p_ref[1] + b_ref[...]
    h = jnp.maximum(s, 0.0) + res_ref[...]
    hn = _ln(h, g_ref[...], be_ref[...])
    o_ref[...] = jnp.dot(hn, wot_ref[...],
                         preferred_element_type=jnp.float32) + bo_ref[...]


_row_spec = pl.BlockSpec((_BN, D), lambda i: (i, 0))
_mat_spec = pl.BlockSpec((D, D), lambda i: (0, 0))
_vec_spec = pl.BlockSpec((1, D), lambda i: (0, 0))
_par_spec = pl.BlockSpec((NC, _BN, D), lambda i: (0, i, 0))
_out2 = [jax.ShapeDtypeStruct((N, D), jnp.float32)] * 2
_out1 = jax.ShapeDtypeStruct((N, D), jnp.float32)


@jax.jit
def _pre(x, wint, bin_, w0):
    return pl.pallas_call(
        _pre_body,
        grid=(N // _BN,),
        in_specs=[_row_spec, _mat_spec, _vec_spec, _mat_spec],
        out_specs=[_row_spec, _row_spec],
        out_shape=_out2,
    )(x, wint, bin_, w0)


@jax.jit
def _mid(p, b, res, g, be, wn):
    return pl.pallas_call(
        _mid_body,
        grid=(N // _BN,),
        in_specs=[_par_spec, _vec_spec, _row_spec, _vec_spec, _vec_spec,
                  _mat_spec],
        out_specs=[_row_spec, _row_spec],
        out_shape=_out2,
    )(p, b, res, g, be, wn)


@jax.jit
def _fin(p, b, res, g, be, wot, bo):
    return pl.pallas_call(
        _fin_body,
        grid=(N // _BN,),
        in_specs=[_par_spec, _vec_spec, _row_spec, _vec_spec, _vec_spec,
                  _mat_spec, _vec_spec],
        out_specs=_row_spec,
        out_shape=_out1,
    )(p, b, res, g, be, wot, bo)


def kernel(node_features, W_in, b_in, W0, b0, g0, be0, W1, b1, g1, be1,
           W2, b2, g2, be2, W_out, b_out, edge_index):
    srcs = edge_index[0]
    dsts = edge_index[1].reshape(NW, NCH, CH)
    zeros = jnp.zeros((CH, D), jnp.float32)

    r2 = lambda v: v.reshape(1, D)

    h, hw = _pre(node_features, W_in.T, r2(b_in), W0)

    p = _aggregate(hw, srcs, dsts, zeros)
    h, hw = _mid(p, r2(b0), h, r2(g0), r2(be0), W1)

    p = _aggregate(hw, srcs, dsts, zeros)
    h, hw = _mid(p, r2(b1), h, r2(g1), r2(be1), W2)

    p = _aggregate(hw, srcs, dsts, zeros)
    return _fin(p, r2(b2), h, r2(g2), r2(be2), W_out.T, r2(b_out))


# speedup vs baseline: 1.2245x; 1.0041x over previous
import functools

import jax
import jax.numpy as jnp
from jax import lax
from jax.experimental import pallas as pl
from jax.experimental.pallas import tpu as pltpu
from jax.experimental.pallas import tpu_sc as plsc

N = 10000
E = 320000
D = 128

NC = 2
NS = 16
NW = NC * NS

EPT = E // NW
CH = 80
NCH = EPT // CH
NPAD = 10240
RPT = NPAD // NS
OC = RPT // CH

_BN = 1000


def _agg_kernel_entry(table, srcs, dsts, zeros, out,
                      sb0, sb1, dst_v, r0, r1, acc,
                      sg0, sg1, ss0, ss1):
    sbuf = [sb0, sb1]
    rows = [r0, r1]
    semg = [sg0, sg1]
    sems = [ss0, ss1]

    cid = lax.axis_index("c")
    sid = lax.axis_index("s")
    tid = cid * NS + sid
    ebase = tid * EPT

    def f_src(j, b):
        off = pl.multiple_of(ebase + j * CH, 8)
        pltpu.async_copy(srcs.at[pl.ds(off, CH)], sbuf[b], sems[b])

    def f_gather(b):
        pltpu.async_copy(table.at[sbuf[b]], rows[b], semg[b])

    def wait_g(b):
        pltpu.make_async_copy(table.at[pl.ds(0, CH)], rows[b], semg[b]).wait()

    def wait_s(b):
        pltpu.make_async_copy(srcs.at[pl.ds(0, CH)], sbuf[b], sems[b]).wait()

    def slab(k):
        return pl.multiple_of(sid * RPT + k * CH, 8)

    with jax.named_scope("agg_zero"):
        dslab = pltpu.async_copy(dsts.at[tid], dst_v, sg0)
        pltpu.sync_copy(zeros, r0)
        for k in range(OC):
            pltpu.async_copy(r0, acc.at[pl.ds(slab(k), CH)], ss0)
        for k in range(OC):
            pltpu.make_async_copy(zeros, r1, ss0).wait()
        dslab.wait()

    plsc.subcore_barrier()

    with jax.named_scope("agg_edges"):
        f_src(0, 0)
        f_src(1, 1)
        wait_s(0)
        f_gather(0)
        wait_s(1)
        f_gather(1)

        def body(t, carry):
            j0 = 2 * t
            wait_g(0)
            f_src(j0 + 2, 0)
            pltpu.sync_copy(r0, acc.at[dst_v.at[j0]], add=True)
            wait_s(0)
            f_gather(0)

            wait_g(1)
            f_src(j0 + 3, 1)
            pltpu.sync_copy(r1, acc.at[dst_v.at[j0 + 1]], add=True)
            wait_s(1)
            f_gather(1)
            return carry

        lax.fori_loop(0, NCH // 2 - 1, body, 0)
        wait_g(0)
        f_src(NCH - 1, 0)
        pltpu.sync_copy(r0, acc.at[dst_v.at[NCH - 3]], add=True)
        wait_s(0)
        f_gather(0)
        wait_g(1)
        pltpu.sync_copy(r1, acc.at[dst_v.at[NCH - 2]], add=True)
        wait_g(0)
        pltpu.sync_copy(r0, acc.at[dst_v.at[NCH - 1]], add=True)
        plsc.subcore_barrier()

    with jax.named_scope("agg_out"):
        def f_rd(k, b):
            pltpu.async_copy(acc.at[pl.ds(slab(k), CH)], rows[b], semg[b])

        def f_wr(k, b):
            pltpu.async_copy(rows[b], out.at[cid, pl.ds(slab(k), CH)],
                             sems[b])

        def wait_wr(b):
            pltpu.make_async_copy(table.at[pl.ds(0, CH)], rows[b],
                                  sems[b]).wait()

        f_rd(0, 0)
        f_rd(1, 1)
        for k in range(OC):
            b = k % 2
            wait_g(b)
            f_wr(k, b)
            if k + 2 < OC:
                wait_wr(b)
                f_rd(k + 2, b)
        wait_wr(0)
        wait_wr(1)


@jax.jit
def _aggregate(table, srcs, dsts, zeros):
    mesh = plsc.VectorSubcoreMesh(core_axis_name="c", subcore_axis_name="s")
    k = functools.partial(
        pl.kernel,
        mesh=mesh,
        out_type=jax.ShapeDtypeStruct((NC, NPAD, D), jnp.float32),
        scratch_types=[
            pltpu.VMEM((CH,), jnp.int32),
            pltpu.VMEM((CH,), jnp.int32),
            pltpu.VMEM((NCH, CH), jnp.int32),
            pltpu.VMEM((CH, D), jnp.float32),
            pltpu.VMEM((CH, D), jnp.float32),
            pltpu.VMEM_SHARED((NPAD, D), jnp.float32),
        ] + [pltpu.SemaphoreType.DMA] * 4,
    )(_agg_kernel_entry)
    return k(table, srcs, dsts, zeros)


def _pre_body(x_ref, wint_ref, bin_ref, w0_ref, h_ref, hw_ref):
    h = jnp.dot(x_ref[...], wint_ref[...],
                preferred_element_type=jnp.float32) + bin_ref[...]
    h_ref[...] = h
    hw_ref[...] = jnp.dot(h, w0_ref[...], preferred_element_type=jnp.float32)


def _ln(h, g, be):
    mu = jnp.mean(h, axis=-1, keepdims=True)
    var = jnp.mean((h - mu) ** 2, axis=-1, keepdims=True)
    return (h - mu) * lax.rsqrt(var + 1e-5) * g + be


def _mid_body(p_ref, b_ref, res_ref, g_ref, be_ref, wn_ref, h_ref, hw_ref):
    s = p_ref[0] + p_ref[1] + b_ref[...]
    h = jnp.maximum(s, 0.0) + res_ref[...]
    hn = _ln(h, g_ref[...], be_ref[...])
    h_ref[...] = hn
    hw_ref[...] = jnp.dot(hn, wn_ref[...], preferred_element_type=jnp.float32)


def _fin_body(p_ref, b_ref, res_ref, g_ref, be_ref, wot_ref, bo_ref, o_ref):
    s = p_ref[0] + p_ref[1] + b_ref[...]
    h = jnp.maximum(s, 0.0) + res_ref[...]
    hn = _ln(h, g_ref[...], be_ref[...])
    o_ref[...] = jnp.dot(hn, wot_ref[...],
                         preferred_element_type=jnp.float32) + bo_ref[...]


_row_spec = pl.BlockSpec((_BN, D), lambda i: (i, 0))
_mat_spec = pl.BlockSpec((D, D), lambda i: (0, 0))
_vec_spec = pl.BlockSpec((1, D), lambda i: (0, 0))
_par_spec = pl.BlockSpec((NC, _BN, D), lambda i: (0, i, 0))
_out2 = [jax.ShapeDtypeStruct((N, D), jnp.float32)] * 2
_out1 = jax.ShapeDtypeStruct((N, D), jnp.float32)


@jax.jit
def _pre(x, wint, bin_, w0):
    return pl.pallas_call(
        _pre_body,
        grid=(N // _BN,),
        in_specs=[_row_spec, _mat_spec, _vec_spec, _mat_spec],
        out_specs=[_row_spec, _row_spec],
        out_shape=_out2,
    )(x, wint, bin_, w0)


@jax.jit
def _mid(p, b, res, g, be, wn):
    return pl.pallas_call(
        _mid_body,
        grid=(N // _BN,),
        in_specs=[_par_spec, _vec_spec, _row_spec, _vec_spec, _vec_spec,
                  _mat_spec],
        out_specs=[_row_spec, _row_spec],
        out_shape=_out2,
    )(p, b, res, g, be, wn)


@jax.jit
def _fin(p, b, res, g, be, wot, bo):
    return pl.pallas_call(
        _fin_body,
        grid=(N // _BN,),
        in_specs=[_par_spec, _vec_spec, _row_spec, _vec_spec, _vec_spec,
                  _mat_spec, _vec_spec],
        out_specs=_row_spec,
        out_shape=_out1,
    )(p, b, res, g, be, wot, bo)


def kernel(node_features, W_in, b_in, W0, b0, g0, be0, W1, b1, g1, be1,
           W2, b2, g2, be2, W_out, b_out, edge_index):
    srcs = edge_index[0]
    dsts = edge_index[1].reshape(NW, NCH, CH)
    zeros = jnp.zeros((CH, D), jnp.float32)

    r2 = lambda v: v.reshape(1, D)

    h, hw = _pre(node_features, W_in.T, r2(b_in), W0)

    p = _aggregate(hw, srcs, dsts, zeros)
    h, hw = _mid(p, r2(b0), h, r2(g0), r2(be0), W1)

    p = _aggregate(hw, srcs, dsts, zeros)
    h, hw = _mid(p, r2(b1), h, r2(g1), r2(be1), W2)

    p = _aggregate(hw, srcs, dsts, zeros)
    return _fin(p, r2(b2), h, r2(g2), r2(be2), W_out.T, r2(b_out))


# speedup vs baseline: 1.2509x; 1.0216x over previous
import functools

import jax
import jax.numpy as jnp
from jax import lax
from jax.experimental import pallas as pl
from jax.experimental.pallas import tpu as pltpu
from jax.experimental.pallas import tpu_sc as plsc

N = 10000
E = 320000
D = 128

NC = 2
NS = 16
NW = NC * NS

EPT = E // NW
CH = 80
NCH = EPT // CH
NPAD = 10240
RPT = NPAD // NS
OC = RPT // CH

_BN = 2000


def _agg_kernel_entry(table, srcs, dsts, zeros, out,
                      sb0, sb1, dst_v, r0, r1, acc,
                      sg0, sg1, ss0, ss1):
    sbuf = [sb0, sb1]
    rows = [r0, r1]
    semg = [sg0, sg1]
    sems = [ss0, ss1]

    cid = lax.axis_index("c")
    sid = lax.axis_index("s")
    tid = cid * NS + sid
    ebase = tid * EPT

    def f_src(j, b):
        off = pl.multiple_of(ebase + j * CH, 8)
        pltpu.async_copy(srcs.at[pl.ds(off, CH)], sbuf[b], sems[b])

    def f_gather(b):
        pltpu.async_copy(table.at[sbuf[b]], rows[b], semg[b])

    def wait_g(b):
        pltpu.make_async_copy(table.at[pl.ds(0, CH)], rows[b], semg[b]).wait()

    def wait_s(b):
        pltpu.make_async_copy(srcs.at[pl.ds(0, CH)], sbuf[b], sems[b]).wait()

    def slab(k):
        return pl.multiple_of(sid * RPT + k * CH, 8)

    with jax.named_scope("agg_zero"):
        dslab = pltpu.async_copy(dsts.at[tid], dst_v, sg0)
        pltpu.sync_copy(zeros, r0)
        for k in range(OC):
            pltpu.async_copy(r0, acc.at[pl.ds(slab(k), CH)], ss0)
        for k in range(OC):
            pltpu.make_async_copy(zeros, r1, ss0).wait()
        dslab.wait()

    plsc.subcore_barrier()

    with jax.named_scope("agg_edges"):
        f_src(0, 0)
        f_src(1, 1)
        wait_s(0)
        f_gather(0)
        wait_s(1)
        f_gather(1)

        def body(t, carry):
            j0 = 2 * t
            wait_g(0)
            f_src(j0 + 2, 0)
            pltpu.sync_copy(r0, acc.at[dst_v.at[j0]], add=True)
            wait_s(0)
            f_gather(0)

            wait_g(1)
            f_src(j0 + 3, 1)
            pltpu.sync_copy(r1, acc.at[dst_v.at[j0 + 1]], add=True)
            wait_s(1)
            f_gather(1)
            return carry

        lax.fori_loop(0, NCH // 2 - 1, body, 0)
        wait_g(0)
        f_src(NCH - 1, 0)
        pltpu.sync_copy(r0, acc.at[dst_v.at[NCH - 3]], add=True)
        wait_s(0)
        f_gather(0)
        wait_g(1)
        pltpu.sync_copy(r1, acc.at[dst_v.at[NCH - 2]], add=True)
        wait_g(0)
        pltpu.sync_copy(r0, acc.at[dst_v.at[NCH - 1]], add=True)
        plsc.subcore_barrier()

    with jax.named_scope("agg_out"):
        def f_rd(k, b):
            pltpu.async_copy(acc.at[pl.ds(slab(k), CH)], rows[b], semg[b])

        def f_wr(k, b):
            pltpu.async_copy(rows[b], out.at[cid, pl.ds(slab(k), CH)],
                             sems[b])

        def wait_wr(b):
            pltpu.make_async_copy(table.at[pl.ds(0, CH)], rows[b],
                                  sems[b]).wait()

        f_rd(0, 0)
        f_rd(1, 1)
        for k in range(OC):
            b = k % 2
            wait_g(b)
            f_wr(k, b)
            if k + 2 < OC:
                wait_wr(b)
                f_rd(k + 2, b)
        wait_wr(0)
        wait_wr(1)


@jax.jit
def _aggregate(table, srcs, dsts, zeros):
    mesh = plsc.VectorSubcoreMesh(core_axis_name="c", subcore_axis_name="s")
    k = functools.partial(
        pl.kernel,
        mesh=mesh,
        out_type=jax.ShapeDtypeStruct((NC, NPAD, D), jnp.float32),
        scratch_types=[
            pltpu.VMEM((CH,), jnp.int32),
            pltpu.VMEM((CH,), jnp.int32),
            pltpu.VMEM((NCH, CH), jnp.int32),
            pltpu.VMEM((CH, D), jnp.float32),
            pltpu.VMEM((CH, D), jnp.float32),
            pltpu.VMEM_SHARED((NPAD, D), jnp.float32),
        ] + [pltpu.SemaphoreType.DMA] * 4,
    )(_agg_kernel_entry)
    return k(table, srcs, dsts, zeros)


def _pre_body(x_ref, wint_ref, bin_ref, w0_ref, h_ref, hw_ref):
    h = jnp.dot(x_ref[...], wint_ref[...],
                preferred_element_type=jnp.float32) + bin_ref[...]
    h_ref[...] = h
    hw_ref[...] = jnp.dot(h, w0_ref[...], preferred_element_type=jnp.float32)


def _ln(h, g, be):
    mu = jnp.mean(h, axis=-1, keepdims=True)
    var = jnp.mean((h - mu) ** 2, axis=-1, keepdims=True)
    return (h - mu) * lax.rsqrt(var + 1e-5) * g + be


def _mid_body(p_ref, b_ref, res_ref, g_ref, be_ref, wn_ref, h_ref, hw_ref):
    s = p_ref[0] + p_ref[1] + b_ref[...]
    h = jnp.maximum(s, 0.0) + res_ref[...]
    hn = _ln(h, g_ref[...], be_ref[...])
    h_ref[...] = hn
    hw_ref[...] = jnp.dot(hn, wn_ref[...], preferred_element_type=jnp.float32)


def _fin_body(p_ref, b_ref, res_ref, g_ref, be_ref, wot_ref, bo_ref, o_ref):
    s = p_ref[0] + p_ref[1] + b_ref[...]
    h = jnp.maximum(s, 0.0) + res_ref[...]
    hn = _ln(h, g_ref[...], be_ref[...])
    o_ref[...] = jnp.dot(hn, wot_ref[...],
                         preferred_element_type=jnp.float32) + bo_ref[...]


_row_spec = pl.BlockSpec((_BN, D), lambda i: (i, 0))
_mat_spec = pl.BlockSpec((D, D), lambda i: (0, 0))
_vec_spec = pl.BlockSpec((1, D), lambda i: (0, 0))
_par_spec = pl.BlockSpec((NC, _BN, D), lambda i: (0, i, 0))
_out2 = [jax.ShapeDtypeStruct((N, D), jnp.float32)] * 2
_out1 = jax.ShapeDtypeStruct((N, D), jnp.float32)


@jax.jit
def _pre(x, wint, bin_, w0):
    return pl.pallas_call(
        _pre_body,
        grid=(N // _BN,),
        in_specs=[_row_spec, _mat_spec, _vec_spec, _mat_spec],
        out_specs=[_row_spec, _row_spec],
        out_shape=_out2,
    )(x, wint, bin_, w0)


@jax.jit
def _mid(p, b, res, g, be, wn):
    return pl.pallas_call(
        _mid_body,
        grid=(N // _BN,),
        in_specs=[_par_spec, _vec_spec, _row_spec, _vec_spec, _vec_spec,
                  _mat_spec],
        out_specs=[_row_spec, _row_spec],
        out_shape=_out2,
    )(p, b, res, g, be, wn)


@jax.jit
def _fin(p, b, res, g, be, wot, bo):
    return pl.pallas_call(
        _fin_body,
        grid=(N // _BN,),
        in_specs=[_par_spec, _vec_spec, _row_spec, _vec_spec, _vec_spec,
                  _mat_spec, _vec_spec],
        out_specs=_row_spec,
        out_shape=_out1,
    )(p, b, res, g, be, wot, bo)


def kernel(node_features, W_in, b_in, W0, b0, g0, be0, W1, b1, g1, be1,
           W2, b2, g2, be2, W_out, b_out, edge_index):
    srcs = edge_index[0]
    dsts = edge_index[1].reshape(NW, NCH, CH)
    zeros = jnp.zeros((CH, D), jnp.float32)

    r2 = lambda v: v.reshape(1, D)

    h, hw = _pre(node_features, W_in.T, r2(b_in), W0)

    p = _aggregate(hw, srcs, dsts, zeros)
    h, hw = _mid(p, r2(b0), h, r2(g0), r2(be0), W1)

    p = _aggregate(hw, srcs, dsts, zeros)
    h, hw = _mid(p, r2(b1), h, r2(g1), r2(be1), W2)

    p = _aggregate(hw, srcs, dsts, zeros)
    return _fin(p, r2(b2), h, r2(g2), r2(be2), W_out.T, r2(b_out))
